# Initial kernel scaffold; baseline (speedup 1.0000x reference)
import jax
import jax.numpy as jnp
from jax.experimental import pallas as pl


def kernel(coordinates, atom_ix, weights, bias):
    raise NotImplementedError("write your pallas kernel here")


# speedup vs baseline: 17552.5556x; 17552.5556x over previous
import jax
import jax.numpy as jnp
from jax.experimental import pallas as pl
from jax.experimental.pallas import tpu as pltpu

COULOMB = -231000.0
N = 4096
T = 118
TP = 128
WPAD = 7040


def _energy_kernel(coords_ref, ai_ref, w_ref, bias_ref, out_ref, m_ref):
    types = jax.lax.broadcasted_iota(jnp.int32, (TP, N), 0)
    ai = jnp.broadcast_to(ai_ref[...], (TP, N))
    onehot = (ai == types).astype(jnp.float32)
    counts = jnp.sum(onehot, axis=1, keepdims=True)

    for ti in range(T):
        s = ti * (ti + 1) // 2
        m_ref[ti : ti + 1, :] = w_ref[0:1, s : s + TP]
    m_ref[T:TP, :] = jnp.zeros((TP - T, TP), jnp.float32)

    mv = jax.lax.dot_general(
        m_ref[...], counts, (((1,), (0,)), ((), ())),
        preferred_element_type=jnp.float32)
    pair_sum = jnp.sum(counts * mv)

    c = coords_ref[...]
    s2 = jnp.sum(c * c)
    cs = jnp.sum(c, axis=0, keepdims=True)
    d2 = 2.0 * N * s2 - 2.0 * jnp.sum(cs * cs)
    d = jnp.sqrt(d2)
    recip = jnp.nan_to_num(1.0 / d, nan=0.0)

    out_ref[...] = COULOMB * pair_sum * recip + bias_ref[...]


def kernel(coordinates, atom_ix, weights, bias):
    ai = atom_ix.astype(jnp.int32).reshape(1, N)
    wp = jnp.zeros((1, WPAD), jnp.float32).at[0, : weights.shape[0]].set(weights)
    out = pl.pallas_call(
        _energy_kernel,
        out_shape=jax.ShapeDtypeStruct((1, 1), jnp.float32),
        scratch_shapes=[pltpu.VMEM((TP, TP), jnp.float32)],
    )(coordinates, ai, wp, bias.reshape(1, 1))
    return out.reshape(1)
